# Initial kernel scaffold; baseline (speedup 1.0000x reference)
import jax
import jax.numpy as jnp
from jax.experimental import pallas as pl


def kernel(inputs, neighbor_indices):
    raise NotImplementedError("write your pallas kernel here")


# speedup vs baseline: 4.7405x; 4.7405x over previous
import functools

import jax
import jax.numpy as jnp
from jax import lax
from jax.experimental import pallas as pl
from jax.experimental.pallas import tpu as pltpu
from jax.experimental.pallas import tpu_sc as plsc

B, L, K, C = 4, 10000, 16, 256
LANES = 16
CHUNK = 8
NCHUNK = L // CHUNK
NW = 32
ZROW = B * L
BASE_CNT = NCHUNK // NW
EXTRA = NCHUNK - BASE_CNT * NW
MAX_CNT = BASE_CNT + 1
NITEM = MAX_CNT * B
NPAIR = NITEM // 2
GROUP = CHUNK * K


def _max_tree(vals):
    while len(vals) > 1:
        vals = [jnp.maximum(a, b) for a, b in zip(vals[::2], vals[1::2])] + (
            [vals[-1]] if len(vals) % 2 else [])
    return vals[0]


def _sc_body(table_hbm, idx_hbm, out_hbm,
             idx_raw, idxb_a, idxb_b, rows_a, rows_b, ostage,
             sem_a, sem_b):
    wid = lax.axis_index("subcore") * 2 + lax.axis_index("core")
    start = wid * BASE_CNT + jnp.minimum(wid, EXTRA)
    count = BASE_CNT + jnp.where(wid < EXTRA, 1, 0)

    def prepare(t, idxb_x, rows_x, sem_x):
        b = lax.rem(t, B)
        chunk = jnp.minimum(start + lax.div(t, B), NCHUNK - 1)

        @pl.when(b == 0)
        def _():
            pltpu.sync_copy(idx_hbm.at[chunk], idx_raw)

        boff = b * L
        for j in range(GROUP // LANES):
            v = idx_raw[pl.ds(j * LANES, LANES)]
            vb = jnp.where(v >= 0, v + boff, jnp.full((LANES,), ZROW, jnp.int32))
            idxb_x[pl.ds(j * LANES, LANES)] = vb
        pltpu.async_copy(table_hbm.at[idxb_x], rows_x, sem_x)

    def wait(idxb_x, rows_x, sem_x):
        pltpu.make_async_copy(table_hbm.at[idxb_x], rows_x, sem_x).wait()

    def compute(t, rows_x):
        b = lax.rem(t, B)
        local = lax.div(t, B)
        chunk = jnp.minimum(start + local, NCHUNK - 1)
        row0 = b * L + chunk * CHUNK

        @pl.loop(0, CHUNK)
        def _(r):
            base = r * K
            for cc in range(C // LANES):
                sl = pl.ds(cc * LANES, LANES)
                vals = [rows_x[base + k, sl] for k in range(K)]
                ostage[r, sl] = _max_tree(vals)

        @pl.when(local < count)
        def _():
            pltpu.sync_copy(ostage, out_hbm.at[pl.ds(row0, CHUNK), :])

    prepare(jnp.int32(0), idxb_a, rows_a, sem_a)

    @pl.loop(0, NPAIR)
    def _(j):
        t0 = j * 2
        wait(idxb_a, rows_a, sem_a)
        prepare(t0 + 1, idxb_b, rows_b, sem_b)
        compute(t0, rows_a)
        wait(idxb_b, rows_b, sem_b)

        @pl.when(j < NPAIR - 1)
        def _():
            prepare(t0 + 2, idxb_a, rows_a, sem_a)

        compute(t0 + 1, rows_b)


def kernel(inputs, neighbor_indices):
    table = jnp.concatenate(
        [inputs.reshape(B * L, C), jnp.zeros((8, C), jnp.float32)], axis=0)
    idx2 = neighbor_indices.reshape(NCHUNK, GROUP)

    mesh = plsc.VectorSubcoreMesh(core_axis_name="core",
                                  subcore_axis_name="subcore")
    k = pl.kernel(
        _sc_body,
        out_type=jax.ShapeDtypeStruct((B * L, C), jnp.float32),
        mesh=mesh,
        scratch_types=[
            pltpu.VMEM((GROUP,), jnp.int32),
            pltpu.VMEM((GROUP,), jnp.int32),
            pltpu.VMEM((GROUP,), jnp.int32),
            pltpu.VMEM((GROUP, C), jnp.float32),
            pltpu.VMEM((GROUP, C), jnp.float32),
            pltpu.VMEM((CHUNK, C), jnp.float32),
            pltpu.SemaphoreType.DMA,
            pltpu.SemaphoreType.DMA,
        ],
    )
    out = k(table, idx2)
    return out.reshape(B, L, C)
